# baseline (speedup 1.0000x reference)
import functools

import jax
import jax.numpy as jnp
from jax import lax
from jax.experimental import pallas as pl
from jax.experimental.pallas import tpu as pltpu
from jax.experimental.pallas import tpu_sc as plsc

VOCAB = 1000
VMAIN = 896
VTAIL = 128
BATCH = 1024
SEQ = 50
B_TOTAL = BATCH * SEQ
NUM_CORES = 2
NUM_SUBCORES = 16
NUM_WORKERS = NUM_CORES * NUM_SUBCORES
B_PER_W = B_TOTAL // NUM_WORKERS
CHUNK = 40
NCHUNK = B_PER_W // CHUNK
NBUF = 2

_mesh = plsc.VectorSubcoreMesh(core_axis_name="c", subcore_axis_name="s")


@functools.partial(
    pl.kernel,
    out_type=(
        jax.ShapeDtypeStruct((B_TOTAL, VMAIN), jnp.float32),
        jax.ShapeDtypeStruct((B_TOTAL, VTAIL), jnp.float32),
    ),
    mesh=_mesh,
    compiler_params=pltpu.CompilerParams(use_tc_tiling_on_sc=True),
    scratch_types=[
        pltpu.VMEM((B_PER_W,), jnp.int32),
        pltpu.VMEM((NBUF, CHUNK, VMAIN), jnp.float32),
        pltpu.VMEM((NBUF, CHUNK, VTAIL), jnp.float32),
        pltpu.SemaphoreType.DMA,
        pltpu.SemaphoreType.DMA,
        pltpu.SemaphoreType.DMA,
        pltpu.SemaphoreType.DMA,
    ],
)
def _embedding_gather(
    main_hbm, tail_hbm, idx_hbm, out_main, out_tail,
    idx_v, rows_m, rows_t, sm0, sm1, st0, st1,
):
    wid = lax.axis_index("s") * NUM_CORES + lax.axis_index("c")
    base = wid * B_PER_W
    sems_m = (sm0, sm1)
    sems_t = (st0, st1)

    pltpu.sync_copy(idx_hbm.at[pl.ds(base, B_PER_W)], idx_v)

    def start_gather(c, b):
        idx = idx_v.at[pl.ds(c * CHUNK, CHUNK)]
        pltpu.async_copy(main_hbm.at[idx], rows_m.at[b], sems_m[b])
        pltpu.async_copy(tail_hbm.at[idx], rows_t.at[b], sems_t[b])

    def wait_gather(c, b):
        idx = idx_v.at[pl.ds(c * CHUNK, CHUNK)]
        pltpu.make_async_copy(main_hbm.at[idx], rows_m.at[b], sems_m[b]).wait()
        pltpu.make_async_copy(tail_hbm.at[idx], rows_t.at[b], sems_t[b]).wait()

    def write_out(c, b):
        pltpu.sync_copy(rows_m.at[b], out_main.at[pl.ds(base + c * CHUNK, CHUNK)])
        pltpu.sync_copy(rows_t.at[b], out_tail.at[pl.ds(base + c * CHUNK, CHUNK)])

    for b in range(NBUF):
        start_gather(b, b)

    @pl.loop(0, NCHUNK - NBUF, step=NBUF)
    def _(g):
        for b in range(NBUF):
            c = g + b
            wait_gather(c, b)
            write_out(c, b)
            start_gather(c + NBUF, b)

    for b in range(NBUF):
        c = NCHUNK - NBUF + b
        wait_gather(c, b)
        write_out(c, b)


def kernel(index, token_embedding_table):
    table_main = token_embedding_table[:, :VMAIN]
    table_tail = token_embedding_table[:, VMAIN:]
    table_tail = jnp.pad(table_tail, ((0, 0), (0, VTAIL - (VOCAB - VMAIN))))
    idx_flat = index.reshape(-1)
    out_main, out_tail = _embedding_gather(table_main, table_tail, idx_flat)
    out = jnp.concatenate(
        [out_main, out_tail[:, : VOCAB - VMAIN]], axis=-1
    ).reshape(BATCH, SEQ, VOCAB)
    return out, None


# speedup vs baseline: 1.2399x; 1.2399x over previous
import functools

import jax
import jax.numpy as jnp
from jax import lax
from jax.experimental import pallas as pl
from jax.experimental.pallas import tpu as pltpu
from jax.experimental.pallas import tpu_sc as plsc

VOCAB = 1000
VMAIN = 896
VTAIL = 128
VREM = VOCAB - VMAIN
BATCH = 1024
SEQ = 50
B_TOTAL = BATCH * SEQ
NUM_CORES = 2
NUM_SUBCORES = 16
NUM_WORKERS = NUM_CORES * NUM_SUBCORES
B_PER_W = B_TOTAL // NUM_WORKERS
CHUNK = 40
NCHUNK = B_PER_W // CHUNK
NBUF = 2
LANES = 16

_mesh = plsc.VectorSubcoreMesh(core_axis_name="c", subcore_axis_name="s")


@functools.partial(
    pl.kernel,
    out_type=jax.ShapeDtypeStruct((B_TOTAL, VOCAB), jnp.float32),
    mesh=_mesh,
    compiler_params=pltpu.CompilerParams(
        use_tc_tiling_on_sc=True, needs_layout_passes=False
    ),
    scratch_types=[
        pltpu.VMEM((B_PER_W,), jnp.int32),
        pltpu.VMEM((NBUF, CHUNK, VOCAB), jnp.float32),
        pltpu.VMEM((NBUF, CHUNK, VTAIL), jnp.float32),
        pltpu.SemaphoreType.DMA,
        pltpu.SemaphoreType.DMA,
        pltpu.SemaphoreType.DMA,
        pltpu.SemaphoreType.DMA,
    ],
)
def _embedding_gather(
    main_hbm, tail_hbm, idx_hbm, out_hbm,
    idx_v, rows_v, rows_t, sm0, sm1, st0, st1,
):
    wid = lax.axis_index("s") * NUM_CORES + lax.axis_index("c")
    base = wid * B_PER_W
    sems_m = (sm0, sm1)
    sems_t = (st0, st1)

    pltpu.sync_copy(idx_hbm.at[pl.ds(base, B_PER_W)], idx_v)

    def start_gather(c, b):
        idx = idx_v.at[pl.ds(c * CHUNK, CHUNK)]
        pltpu.async_copy(
            main_hbm.at[idx], rows_v.at[b].at[:, pl.ds(0, VMAIN)], sems_m[b]
        )
        pltpu.async_copy(tail_hbm.at[idx], rows_t.at[b], sems_t[b])

    def wait_gather(c, b):
        idx = idx_v.at[pl.ds(c * CHUNK, CHUNK)]
        pltpu.make_async_copy(
            main_hbm.at[idx], rows_v.at[b].at[:, pl.ds(0, VMAIN)], sems_m[b]
        ).wait()
        pltpu.make_async_copy(tail_hbm.at[idx], rows_t.at[b], sems_t[b]).wait()

    def repack_tail(b):
        lane = lax.iota(jnp.int32, LANES)
        rem_cols = VMAIN + (VREM // LANES) * LANES + lane
        rem_mask = rem_cols < VOCAB

        @pl.loop(0, CHUNK)
        def _(r):
            for k in range(VREM // LANES):
                rows_v.at[b][r, pl.ds(VMAIN + k * LANES, LANES)] = (
                    rows_t.at[b][r, pl.ds(k * LANES, LANES)]
                )
            x = rows_t.at[b][r, pl.ds((VREM // LANES) * LANES, LANES)]
            row_ids = jnp.full((LANES,), r, jnp.int32)
            plsc.store_scatter(
                rows_v.at[b], [row_ids, rem_cols], x, mask=rem_mask
            )

    def write_out(c, b):
        pltpu.sync_copy(rows_v.at[b], out_hbm.at[pl.ds(base + c * CHUNK, CHUNK)])

    for b in range(NBUF):
        start_gather(b, b)

    @pl.loop(0, NCHUNK - NBUF, step=NBUF)
    def _(g):
        for b in range(NBUF):
            c = g + b
            wait_gather(c, b)
            repack_tail(b)
            write_out(c, b)
            start_gather(c + NBUF, b)

    for b in range(NBUF):
        c = NCHUNK - NBUF + b
        wait_gather(c, b)
        repack_tail(b)
        write_out(c, b)


def kernel(index, token_embedding_table):
    table_main = token_embedding_table[:, :VMAIN]
    table_tail = jnp.pad(
        token_embedding_table[:, VMAIN:], ((0, 0), (0, VTAIL - VREM))
    )
    idx_flat = index.reshape(-1)
    out = _embedding_gather(table_main, table_tail, idx_flat)
    return out.reshape(BATCH, SEQ, VOCAB), None
